# baseline (speedup 1.0000x reference)
import jax
import jax.numpy as jnp
from jax.experimental import pallas as pl

_NUM_EXPERTS = 8
_BT = 2048


def _router_block(x_ref, w_ref, logits_ref, prob_ref, idx_ref):
    x = x_ref[...]
    w = w_ref[...]
    logits = jax.lax.dot_general(
        x, w, (((1,), (1,)), ((), ())), preferred_element_type=jnp.float32
    )
    logits_ref[...] = logits

    lane = jax.lax.broadcasted_iota(jnp.int32, logits.shape, 1)
    m1 = jnp.max(logits, axis=1, keepdims=True)
    i1 = jnp.min(jnp.where(logits == m1, lane, _NUM_EXPERTS), axis=1, keepdims=True)
    masked = jnp.where(lane == i1, -jnp.inf, logits)
    m2 = jnp.max(masked, axis=1, keepdims=True)
    i2 = jnp.min(jnp.where(masked == m2, lane, _NUM_EXPERTS), axis=1, keepdims=True)

    e = jnp.exp(m2 - m1)
    denom = 1.0 + e
    p1 = 1.0 / denom
    p2 = e / denom
    prob_ref[...] = jnp.concatenate([p1, p2], axis=1)
    idx_ref[...] = jnp.concatenate([i1, i2], axis=1)


def kernel(hidden_states, weight):
    n_tokens, hidden = hidden_states.shape
    n_experts = weight.shape[0]
    grid = (n_tokens // _BT,)
    out = pl.pallas_call(
        _router_block,
        grid=grid,
        in_specs=[
            pl.BlockSpec((_BT, hidden), lambda i: (i, 0)),
            pl.BlockSpec((n_experts, hidden), lambda i: (0, 0)),
        ],
        out_specs=[
            pl.BlockSpec((_BT, n_experts), lambda i: (i, 0)),
            pl.BlockSpec((_BT, 2), lambda i: (i, 0)),
            pl.BlockSpec((_BT, 2), lambda i: (i, 0)),
        ],
        out_shape=[
            jax.ShapeDtypeStruct((n_tokens, n_experts), jnp.float32),
            jax.ShapeDtypeStruct((n_tokens, 2), jnp.float32),
            jax.ShapeDtypeStruct((n_tokens, 2), jnp.int32),
        ],
    )(hidden_states, weight)
    logits, topk_prob, topk_idx = out
    return (logits, topk_prob, topk_idx)


# speedup vs baseline: 1.6945x; 1.6945x over previous
import jax
import jax.numpy as jnp
from jax.experimental import pallas as pl

_NUM_EXPERTS = 8
_BT = 2048


def _router_block(x_ref, w_ref, logits_ref, prob_ref, idx_ref):
    x = x_ref[...]
    w = w_ref[...]
    logits = jax.lax.dot_general(
        x, w, (((1,), (1,)), ((), ())), preferred_element_type=jnp.float32
    )
    logits_ref[...] = logits

    lt = logits.T
    sub = jax.lax.broadcasted_iota(jnp.int32, lt.shape, 0)
    m1 = jnp.max(lt, axis=0, keepdims=True)
    i1 = jnp.min(jnp.where(lt == m1, sub, _NUM_EXPERTS), axis=0, keepdims=True)
    masked = jnp.where(sub == i1, -jnp.inf, lt)
    m2 = jnp.max(masked, axis=0, keepdims=True)
    i2 = jnp.min(jnp.where(masked == m2, sub, _NUM_EXPERTS), axis=0, keepdims=True)

    e = jnp.exp(m2 - m1)
    denom = 1.0 + e
    prob_ref[...] = jnp.concatenate([1.0 / denom, e / denom], axis=0)
    idx_ref[...] = jnp.concatenate([i1, i2], axis=0)


def kernel(hidden_states, weight):
    n_tokens, hidden = hidden_states.shape
    n_experts = weight.shape[0]
    grid = (n_tokens // _BT,)
    logits, prob_t, idx_t = pl.pallas_call(
        _router_block,
        grid=grid,
        in_specs=[
            pl.BlockSpec((_BT, hidden), lambda i: (i, 0)),
            pl.BlockSpec((n_experts, hidden), lambda i: (0, 0)),
        ],
        out_specs=[
            pl.BlockSpec((_BT, n_experts), lambda i: (i, 0)),
            pl.BlockSpec((2, _BT), lambda i: (0, i)),
            pl.BlockSpec((2, _BT), lambda i: (0, i)),
        ],
        out_shape=[
            jax.ShapeDtypeStruct((n_tokens, n_experts), jnp.float32),
            jax.ShapeDtypeStruct((2, n_tokens), jnp.float32),
            jax.ShapeDtypeStruct((2, n_tokens), jnp.int32),
        ],
    )(hidden_states, weight)
    return (logits, prob_t.T, idx_t.T)


# speedup vs baseline: 1.7345x; 1.0236x over previous
import jax
import jax.numpy as jnp
from jax.experimental import pallas as pl

_NUM_EXPERTS = 8
_BT = 4096


def _router_block(x_ref, w_ref, logits_ref, prob_ref, idx_ref):
    x = x_ref[...]
    w = w_ref[...]
    logits = jax.lax.dot_general(
        x, w, (((1,), (1,)), ((), ())), preferred_element_type=jnp.float32
    )
    logits_ref[...] = logits

    lt = logits.T
    sub = jax.lax.broadcasted_iota(jnp.int32, lt.shape, 0)
    m1 = jnp.max(lt, axis=0, keepdims=True)
    i1 = jnp.min(jnp.where(lt == m1, sub, _NUM_EXPERTS), axis=0, keepdims=True)
    masked = jnp.where(sub == i1, -jnp.inf, lt)
    m2 = jnp.max(masked, axis=0, keepdims=True)
    i2 = jnp.min(jnp.where(masked == m2, sub, _NUM_EXPERTS), axis=0, keepdims=True)

    e = jnp.exp(m2 - m1)
    denom = 1.0 + e
    prob_ref[...] = jnp.concatenate([1.0 / denom, e / denom], axis=0)
    idx_ref[...] = jnp.concatenate([i1, i2], axis=0)


def kernel(hidden_states, weight):
    n_tokens, hidden = hidden_states.shape
    n_experts = weight.shape[0]
    grid = (n_tokens // _BT,)
    logits, prob_t, idx_t = pl.pallas_call(
        _router_block,
        grid=grid,
        in_specs=[
            pl.BlockSpec((_BT, hidden), lambda i: (i, 0)),
            pl.BlockSpec((n_experts, hidden), lambda i: (0, 0)),
        ],
        out_specs=[
            pl.BlockSpec((_BT, n_experts), lambda i: (i, 0)),
            pl.BlockSpec((2, _BT), lambda i: (0, i)),
            pl.BlockSpec((2, _BT), lambda i: (0, i)),
        ],
        out_shape=[
            jax.ShapeDtypeStruct((n_tokens, n_experts), jnp.float32),
            jax.ShapeDtypeStruct((2, n_tokens), jnp.float32),
            jax.ShapeDtypeStruct((2, n_tokens), jnp.int32),
        ],
    )(hidden_states, weight)
    return (logits, prob_t.T, idx_t.T)
